# Initial kernel scaffold; baseline (speedup 1.0000x reference)
import jax
import jax.numpy as jnp
from jax.experimental import pallas as pl


def kernel(x, emb_tables, linear_w, bias):
    raise NotImplementedError("write your pallas kernel here")


# speedup vs baseline: 7.9840x; 7.9840x over previous
import functools

import jax
import jax.numpy as jnp
from jax import lax
from jax.experimental import pallas as pl
from jax.experimental.pallas import tpu as pltpu
from jax.experimental.pallas import tpu_sc as plsc

F = 26
V = 100000
D = 16
B = 4096
NC = 2
NS = 16
NW = NC * NS
B_PER_W = B // NW
N_CHUNK = 8
CHUNK = 85
IDX_W = N_CHUNK * CHUNK
LIN_W = 32


def _ffm_body(idx_hbm, lidx_hbm, emb_hbm, lin_hbm, bias_hbm, out_hbm,
              idx_v, rows_v, lidx_v, lin_v, bias_v, out_v, sem, lsem):
    wid = lax.axis_index("s") * NC + lax.axis_index("c")
    base = wid * B_PER_W
    pltpu.sync_copy(bias_hbm, bias_v)
    bias_s = bias_v[pl.ds(0, 16)][0]
    lanes = lax.iota(jnp.int32, 16)

    def body(bi, totals):
        b = base + bi
        pltpu.sync_copy(idx_hbm.at[b], idx_v)
        pltpu.sync_copy(lidx_hbm.at[b], lidx_v)
        copies = [
            pltpu.async_copy(emb_hbm.at[idx_v.at[c]],
                             rows_v.at[pl.ds(c * CHUNK, CHUNK)], sem)
            for c in range(N_CHUNK)
        ]
        lcopy = pltpu.async_copy(lin_hbm.at[lidx_v], lin_v, lsem)
        for cp in copies:
            cp.wait()
        lcopy.wait()

        accs = [jnp.zeros((D,), jnp.float32) for _ in range(4)]
        k = 0
        for i in range(F):
            for j in range(i + 1, F):
                a = rows_v[i * F + j]
                bb = rows_v[j * F + i]
                accs[k % 4] = accs[k % 4] + a * bb
                k += 1
        tv = (accs[0] + accs[1] + accs[2] + accs[3]
              + lin_v[pl.ds(0, 16)] + lin_v[pl.ds(16, 16)])
        for sh in (8, 4, 2, 1):
            tv = tv + tv.at[lanes ^ sh].get(mode="promise_in_bounds")
        tv = tv + bias_s
        t = lax.rem(bi, 16)
        totals = jnp.where(lanes == t, tv, totals)

        @pl.when(t == 15)
        def _():
            out_v[pl.ds(bi - 15, 16)] = totals

        return totals

    lax.fori_loop(0, B_PER_W, body, jnp.zeros((16,), jnp.float32))

    for k in range(B_PER_W // 16):
        v = out_v[pl.ds(k * 16, 16)]
        out_v[pl.ds(k * 16, 16)] = 1.0 / (1.0 + jnp.exp(-v))
    pltpu.sync_copy(out_v, out_hbm.at[pl.ds(base, B_PER_W)])


@jax.jit
def _ffm_sc(idx, lidx, emb_flat, lin_pad, bias16):
    run = pl.kernel(
        _ffm_body,
        out_type=jax.ShapeDtypeStruct((B,), jnp.float32),
        mesh=plsc.VectorSubcoreMesh(core_axis_name="c", subcore_axis_name="s"),
        compiler_params=pltpu.CompilerParams(use_tc_tiling_on_sc=False),
        scratch_types=[
            pltpu.VMEM((N_CHUNK, CHUNK), jnp.int32),
            pltpu.VMEM((IDX_W, D), jnp.float32),
            pltpu.VMEM((LIN_W,), jnp.int32),
            pltpu.VMEM((LIN_W,), jnp.float32),
            pltpu.VMEM((16,), jnp.float32),
            pltpu.VMEM((B_PER_W,), jnp.float32),
            pltpu.SemaphoreType.DMA,
            pltpu.SemaphoreType.DMA,
        ],
    )
    return run(idx, lidx, emb_flat, lin_pad, bias16)


def kernel(x, emb_tables, linear_w, bias):
    offs = (jnp.arange(F, dtype=jnp.int32) * V)[None, None, :]
    idx = (x[:, :, None] + offs).reshape(B, F * F)
    idx = jnp.pad(idx, ((0, 0), (0, IDX_W - F * F)))
    idx = idx.reshape(B, N_CHUNK, CHUNK)
    lidx = jnp.pad(x, ((0, 0), (0, LIN_W - F)), constant_values=V)
    lin_pad = jnp.concatenate([linear_w.reshape(V), jnp.zeros((1,), jnp.float32)])
    bias16 = jnp.pad(bias.astype(jnp.float32), (0, 15))
    emb_flat = emb_tables.reshape(F * V, D)
    out = _ffm_sc(idx, lidx, emb_flat, lin_pad, bias16)
    return out.reshape(B, 1)


# speedup vs baseline: 9.4579x; 1.1846x over previous
import functools

import jax
import jax.numpy as jnp
from jax import lax
from jax.experimental import pallas as pl
from jax.experimental.pallas import tpu as pltpu
from jax.experimental.pallas import tpu_sc as plsc

F = 26
V = 100000
D = 16
B = 4096
NC = 2
NS = 16
NW = NC * NS
B_PER_W = B // NW
CHUNK = 85
NCH_ROW = 8
IDX_W = NCH_ROW * CHUNK
LIN_W = 32
G = 4
NG = B_PER_W // G
GCH = G * NCH_ROW
GIDX = G * IDX_W
GLIN = G * LIN_W


def _ffm_body(idx_hbm, lidx_hbm, emb_hbm, lin_hbm, bias_hbm, out_hbm,
              idx_v, rows_v, lidx_v, lin_v, bias_v, out_v, sem0, sem1):
    wid = lax.axis_index("s") * NC + lax.axis_index("c")
    gbase = wid * NG
    pltpu.sync_copy(bias_hbm, bias_v)
    bias_s = bias_v[pl.ds(0, 16)][0]
    lanes = lax.iota(jnp.int32, 16)
    sems = (sem0, sem1)

    def stage(g, slot):
        grow = gbase + g
        pltpu.sync_copy(idx_hbm.at[grow], idx_v.at[pl.ds(slot * GCH, GCH)])
        pltpu.sync_copy(lidx_hbm.at[grow], lidx_v.at[pl.ds(slot * GLIN, GLIN)])
        for c in range(GCH):
            pltpu.async_copy(
                emb_hbm.at[idx_v.at[slot * GCH + c]],
                rows_v.at[pl.ds((slot * GCH + c) * CHUNK, CHUNK)],
                sems[slot])
        pltpu.async_copy(lin_hbm.at[lidx_v.at[pl.ds(slot * GLIN, GLIN)]],
                         lin_v.at[pl.ds(slot * GLIN, GLIN)], sems[slot])

    def wait_slot(slot):
        pltpu.make_async_copy(emb_hbm.at[pl.ds(0, GIDX)],
                              rows_v.at[pl.ds(slot * GIDX, GIDX)],
                              sems[slot]).wait()
        pltpu.make_async_copy(lin_hbm.at[pl.ds(0, GLIN)],
                              lin_v.at[pl.ds(slot * GLIN, GLIN)],
                              sems[slot]).wait()

    def compute(g, slot_off):
        def row_body(r, carry):
            rbase = slot_off * GIDX + r * IDX_W
            lbase = slot_off * GLIN + r * LIN_W
            accs = [jnp.zeros((D,), jnp.float32) for _ in range(4)]
            k = 0
            for i in range(F):
                for j in range(i + 1, F):
                    a = rows_v[rbase + i * F + j]
                    bb = rows_v[rbase + j * F + i]
                    accs[k % 4] = accs[k % 4] + a * bb
                    k += 1
            tv = (accs[0] + accs[1] + accs[2] + accs[3]
                  + lin_v[pl.ds(lbase, 16)] + lin_v[pl.ds(lbase + 16, 16)])
            for sh in (8, 4, 2, 1):
                tv = tv + tv.at[lanes ^ sh].get(mode="promise_in_bounds")
            tv = tv + bias_s
            bi = g * G + r
            off16 = (bi // 16) * 16
            cur = out_v[pl.ds(off16, 16)]
            out_v[pl.ds(off16, 16)] = jnp.where(lanes == bi % 16, tv, cur)
            return carry

        lax.fori_loop(0, G, row_body, 0)

    stage(0, 0)

    def body(g, carry):
        even = lax.rem(g, 2) == 0

        @pl.when(even)
        def _():
            stage(g + 1, 1)

        @pl.when(jnp.logical_and(jnp.logical_not(even), g < NG - 1))
        def _():
            stage(g + 1, 0)

        @pl.when(even)
        def _():
            wait_slot(0)

        @pl.when(jnp.logical_not(even))
        def _():
            wait_slot(1)

        compute(g, lax.rem(g, 2))
        return carry

    lax.fori_loop(0, NG, body, 0)

    for k in range(B_PER_W // 16):
        v = out_v[pl.ds(k * 16, 16)]
        out_v[pl.ds(k * 16, 16)] = 1.0 / (1.0 + jnp.exp(-v))
    pltpu.sync_copy(out_v, out_hbm.at[pl.ds(wid * B_PER_W, B_PER_W)])


@jax.jit
def _ffm_sc(idx, lidx, emb_flat, lin_pad, bias16):
    run = pl.kernel(
        _ffm_body,
        out_type=jax.ShapeDtypeStruct((B,), jnp.float32),
        mesh=plsc.VectorSubcoreMesh(core_axis_name="c", subcore_axis_name="s"),
        compiler_params=pltpu.CompilerParams(use_tc_tiling_on_sc=False),
        scratch_types=[
            pltpu.VMEM((2 * GCH, CHUNK), jnp.int32),
            pltpu.VMEM((2 * GIDX, D), jnp.float32),
            pltpu.VMEM((2 * GLIN,), jnp.int32),
            pltpu.VMEM((2 * GLIN,), jnp.float32),
            pltpu.VMEM((16,), jnp.float32),
            pltpu.VMEM((B_PER_W,), jnp.float32),
            pltpu.SemaphoreType.DMA,
            pltpu.SemaphoreType.DMA,
        ],
    )
    return run(idx, lidx, emb_flat, lin_pad, bias16)


def kernel(x, emb_tables, linear_w, bias):
    offs = (jnp.arange(F, dtype=jnp.int32) * V)[None, None, :]
    idx = (x[:, :, None] + offs).reshape(B, F * F)
    idx = jnp.pad(idx, ((0, 0), (0, IDX_W - F * F)))
    idx = idx.reshape(B // G, GCH, CHUNK)
    lidx = jnp.pad(x, ((0, 0), (0, LIN_W - F)), constant_values=V)
    lidx = lidx.reshape(B // G, GLIN)
    lin_pad = jnp.concatenate([linear_w.reshape(V), jnp.zeros((1,), jnp.float32)])
    bias16 = jnp.pad(bias.astype(jnp.float32), (0, 15))
    emb_flat = emb_tables.reshape(F * V, D)
    out = _ffm_sc(idx, lidx, emb_flat, lin_pad, bias16)
    return out.reshape(B, 1)


# speedup vs baseline: 9.7297x; 1.0287x over previous
import functools

import jax
import jax.numpy as jnp
from jax import lax
from jax.experimental import pallas as pl
from jax.experimental.pallas import tpu as pltpu
from jax.experimental.pallas import tpu_sc as plsc

F = 26
V = 100000
D = 16
B = 4096
FD = F * D
NC = 2
NS = 16
NW = NC * NS
B_PER_W = B // NW
G = 4
GB = G * F
NG = B_PER_W // G
LIN_W = 32
GLIN = G * LIN_W


def _ffm_body(idx_hbm, lidx_hbm, emb_hbm, lin_hbm, bias_hbm, out_hbm,
              idx_v, rows_v, lidx_v, lin_v, bias_v, out_v, sem0, sem1):
    wid = lax.axis_index("s") * NC + lax.axis_index("c")
    gbase = wid * NG
    pltpu.sync_copy(bias_hbm, bias_v)
    bias_s = bias_v[pl.ds(0, 16)][0]
    lanes = lax.iota(jnp.int32, 16)
    sems = (sem0, sem1)

    def stage(g, slot):
        grow = gbase + g
        pltpu.sync_copy(idx_hbm.at[grow], idx_v.at[slot])
        pltpu.sync_copy(lidx_hbm.at[grow], lidx_v.at[pl.ds(slot * GLIN, GLIN)])
        pltpu.async_copy(emb_hbm.at[idx_v.at[slot]],
                         rows_v.at[pl.ds(slot * GB, GB)], sems[slot])
        pltpu.async_copy(lin_hbm.at[lidx_v.at[pl.ds(slot * GLIN, GLIN)]],
                         lin_v.at[pl.ds(slot * GLIN, GLIN)], sems[slot])

    def wait_slot(slot):
        pltpu.make_async_copy(emb_hbm.at[pl.ds(0, GB)],
                              rows_v.at[pl.ds(slot * GB, GB)],
                              sems[slot]).wait()
        pltpu.make_async_copy(lin_hbm.at[pl.ds(0, GLIN)],
                              lin_v.at[pl.ds(slot * GLIN, GLIN)],
                              sems[slot]).wait()

    def compute(g, slot_off):
        def row_body(r, carry):
            rbase = slot_off * GB + r * F
            lbase = slot_off * GLIN + r * LIN_W
            accs = [jnp.zeros((D,), jnp.float32) for _ in range(4)]
            k = 0
            for i in range(F):
                for j in range(i + 1, F):
                    a = rows_v[rbase + i, pl.ds(j * D, D)]
                    bb = rows_v[rbase + j, pl.ds(i * D, D)]
                    accs[k % 4] = accs[k % 4] + a * bb
                    k += 1
            tv = (accs[0] + accs[1] + accs[2] + accs[3]
                  + lin_v[pl.ds(lbase, 16)] + lin_v[pl.ds(lbase + 16, 16)])
            for sh in (8, 4, 2, 1):
                tv = tv + tv.at[lanes ^ sh].get(mode="promise_in_bounds")
            tv = tv + bias_s
            bi = g * G + r
            off16 = (bi // 16) * 16
            cur = out_v[pl.ds(off16, 16)]
            out_v[pl.ds(off16, 16)] = jnp.where(lanes == bi % 16, tv, cur)
            return carry

        lax.fori_loop(0, G, row_body, 0)

    stage(0, 0)

    def body(g, carry):
        even = lax.rem(g, 2) == 0

        @pl.when(even)
        def _():
            stage(g + 1, 1)

        @pl.when(jnp.logical_and(jnp.logical_not(even), g < NG - 1))
        def _():
            stage(g + 1, 0)

        @pl.when(even)
        def _():
            wait_slot(0)

        @pl.when(jnp.logical_not(even))
        def _():
            wait_slot(1)

        compute(g, lax.rem(g, 2))
        return carry

    lax.fori_loop(0, NG, body, 0)

    for k in range(B_PER_W // 16):
        v = out_v[pl.ds(k * 16, 16)]
        out_v[pl.ds(k * 16, 16)] = 1.0 / (1.0 + jnp.exp(-v))
    pltpu.sync_copy(out_v, out_hbm.at[pl.ds(wid * B_PER_W, B_PER_W)])


@jax.jit
def _ffm_sc(x, emb_tables, linear_w, bias):
    emb2 = emb_tables.transpose(1, 0, 2).reshape(V, FD)
    idx = x.astype(jnp.int32).reshape(B // G, GB)
    lidx = jnp.pad(x.astype(jnp.int32), ((0, 0), (0, LIN_W - F)),
                   constant_values=V)
    lidx = lidx.reshape(B // G, GLIN)
    lin_pad = jnp.concatenate([linear_w.reshape(V),
                               jnp.zeros((1,), jnp.float32)])
    bias16 = jnp.pad(bias.astype(jnp.float32).reshape(1), (0, 15))
    run = pl.kernel(
        _ffm_body,
        out_type=jax.ShapeDtypeStruct((B,), jnp.float32),
        mesh=plsc.VectorSubcoreMesh(core_axis_name="c", subcore_axis_name="s"),
        compiler_params=pltpu.CompilerParams(use_tc_tiling_on_sc=False),
        scratch_types=[
            pltpu.VMEM((2, GB), jnp.int32),
            pltpu.VMEM((2 * GB, FD), jnp.float32),
            pltpu.VMEM((2 * GLIN,), jnp.int32),
            pltpu.VMEM((2 * GLIN,), jnp.float32),
            pltpu.VMEM((16,), jnp.float32),
            pltpu.VMEM((B_PER_W,), jnp.float32),
            pltpu.SemaphoreType.DMA,
            pltpu.SemaphoreType.DMA,
        ],
    )
    return run(idx, lidx, emb2, lin_pad, bias16)


def kernel(x, emb_tables, linear_w, bias):
    out = _ffm_sc(x, emb_tables, linear_w, bias)
    return out.reshape(B, 1)
